# Initial kernel scaffold; baseline (speedup 1.0000x reference)
import jax
import jax.numpy as jnp
from jax.experimental import pallas as pl


def kernel(x, gate_W, gate_b, W1, b1, W2, b2, cls_W, cls_b, vec_W, vec_b):
    raise NotImplementedError("write your pallas kernel here")


# speedup vs baseline: 1.0872x; 1.0872x over previous
import functools

import jax
import jax.numpy as jnp
from jax.experimental import pallas as pl
from jax.experimental.pallas import tpu as pltpu

_N = 4096
_D = 2048
_E = 8
_H = 256
_O = 2048
_LANES = 128
_BN = 256


def _gate_weights(x, gwt, gb):
    logits = jax.lax.dot_general(x, gwt, (((1,), (1,)), ((), ())))
    lane = jax.lax.broadcasted_iota(jnp.int32, logits.shape, 1)
    valid = lane < _E
    l = jnp.where(valid, logits + gb, -jnp.inf)
    m = jnp.max(l, axis=1, keepdims=True)
    p = jnp.where(valid, jnp.exp(l - m), 0.0)
    probs = p / jnp.sum(p, axis=1, keepdims=True)
    m1 = jnp.max(probs, axis=1, keepdims=True)
    a1 = jnp.min(jnp.where((probs == m1) & valid, lane, _LANES), axis=1, keepdims=True)
    probs2 = jnp.where(lane == a1, -1.0, probs)
    m2 = jnp.max(probs2, axis=1, keepdims=True)
    a2 = jnp.min(jnp.where((probs2 == m2) & valid, lane, _LANES), axis=1, keepdims=True)
    denom = m1 + m2
    w = jnp.where(lane == a1, m1 / denom, 0.0) + jnp.where(lane == a2, m2 / denom, 0.0)
    return w


def _moe_body(x_ref, gwt_ref, gb_ref, w1_ref, b1_ref, w2_ref, b2_ref, out_ref):
    x = x_ref[...]
    w = _gate_weights(x, gwt_ref[...], gb_ref[...])
    acc = jnp.zeros((_BN, _O), dtype=jnp.float32)
    for e in range(_E):
        h = jax.lax.dot_general(x, w1_ref[e], (((1,), (1,)), ((), ())))
        h = jax.nn.relu(h + b1_ref[e][None, :])
        w_col = w[:, e : e + 1]
        acc = acc + jax.lax.dot_general(h * w_col, w2_ref[e], (((1,), (1,)), ((), ())))
        acc = acc + w_col * b2_ref[e][None, :]
    out_ref[...] = acc


def _heads_body(eo_ref, cw_ref, cb_ref, vw_ref, vb_ref, cls_ref, vec_ref):
    eo = eo_ref[...]
    cls_ref[...] = jax.lax.dot_general(eo, cw_ref[...], (((1,), (1,)), ((), ()))) + cb_ref[...]
    vec_ref[...] = jax.lax.dot_general(eo, vw_ref[...], (((1,), (1,)), ((), ()))) + vb_ref[...]


def kernel(x, gate_W, gate_b, W1, b1, W2, b2, cls_W, cls_b, vec_W, vec_b):
    x = x.astype(jnp.float32)
    gwt = jnp.pad(gate_W, ((0, _LANES - _E), (0, 0)))
    gb = jnp.pad(gate_b, (0, _LANES - _E))[None, :]

    grid = _N // _BN
    whole = lambda shape: pl.BlockSpec(shape, lambda i: (0,) * len(shape))

    eo = pl.pallas_call(
        _moe_body,
        grid=(grid,),
        in_specs=[
            pl.BlockSpec((_BN, _D), lambda i: (i, 0)),
            whole((_LANES, _D)),
            whole((1, _LANES)),
            whole((_E, _H, _D)),
            whole((_E, _H)),
            whole((_E, _O, _H)),
            whole((_E, _O)),
        ],
        out_specs=pl.BlockSpec((_BN, _O), lambda i: (i, 0)),
        out_shape=jax.ShapeDtypeStruct((_N, _O), jnp.float32),
        compiler_params=pltpu.CompilerParams(
            vmem_limit_bytes=110 * 1024 * 1024,
        ),
    )(x, gwt, gb, W1, b1, W2, b2)

    cls_out, vec_out = pl.pallas_call(
        _heads_body,
        grid=(grid,),
        in_specs=[
            pl.BlockSpec((_BN, _O), lambda i: (i, 0)),
            whole((_O, _O)),
            whole((1, _O)),
            whole((_O, _O)),
            whole((1, _O)),
        ],
        out_specs=[
            pl.BlockSpec((_BN, _O), lambda i: (i, 0)),
            pl.BlockSpec((_BN, _O), lambda i: (i, 0)),
        ],
        out_shape=[
            jax.ShapeDtypeStruct((_N, _O), jnp.float32),
            jax.ShapeDtypeStruct((_N, _O), jnp.float32),
        ],
        compiler_params=pltpu.CompilerParams(
            vmem_limit_bytes=110 * 1024 * 1024,
        ),
    )(eo, cls_W, cls_b[None, :], vec_W, vec_b[None, :])

    return (cls_out, vec_out)
